# baseline (speedup 1.0000x reference)
import functools

import jax
import jax.numpy as jnp
from jax import lax
from jax.experimental import pallas as pl
from jax.experimental.pallas import tpu as pltpu
from jax.experimental.pallas import tpu_sc as plsc

R = 16
N = 100000
B = 16384
REG_PARA = 0.01
NC, NS, L = 2, 16, 16
NW = NC * NS
PER_W = B // NW
C = 64
NCH = PER_W // C


def _sc_body(idx_in, w0_in, w1_in, w2_in, p0_in, p1_in, p2_in, out, partials,
             ixall, ixb0, ixb1, ixb2, idx2,
             r0w, r0p, r1w, r1p, r2w, r2p,
             obuf, regbuf, sem):
    wid = lax.axis_index("c") * NS + lax.axis_index("s")
    base = wid * PER_W

    zeros = jnp.zeros((L,), jnp.float32)
    lanes = lax.iota(jnp.int32, L)
    col0 = lanes * 0
    ramp = lanes * N

    pltpu.sync_copy(idx_in.at[pl.ds(base, PER_W)], ixall)

    def decol(g, carry):
        rows = g * L + lanes
        ixb0[pl.ds(g * L, L)] = plsc.load_gather(ixall, [rows, col0])
        ixb1[pl.ds(g * L, L)] = plsc.load_gather(ixall, [rows, col0 + 1])
        ixb2[pl.ds(g * L, L)] = plsc.load_gather(ixall, [rows, col0 + 2])
        return carry
    lax.fori_loop(0, PER_W // L, decol, 0)

    def subchunk(j, carry):
        s0, s1, s2 = carry
        o = j * C
        i0 = ixb0.at[pl.ds(o, C)]
        i1 = ixb1.at[pl.ds(o, C)]

        def build2(g, c2):
            vex = ixb2[pl.ds(o + g * L, L)]
            for i in range(L):
                idx2[pl.ds((g * L + i) * L, L)] = vex[i] + ramp
            return c2
        lax.fori_loop(0, C // L, build2, 0)

        pending = []

        def issue(cp):
            pending.append(cp)
            if len(pending) > 30:
                pending.pop(0).wait()

        issue(pltpu.async_copy(w0_in.at[0].at[i0], r0w, sem))
        issue(pltpu.async_copy(p0_in.at[0].at[i0], r0p, sem))
        for r in range(R):
            sl = pl.ds(r * C, C)
            issue(pltpu.async_copy(w1_in.at[r].at[i1], r1w.at[sl], sem))
            issue(pltpu.async_copy(p1_in.at[r].at[i1], r1p.at[sl], sem))
        for k in range(C * R // 128):
            sl = pl.ds(k * 128, 128)
            issue(pltpu.async_copy(w2_in.at[idx2.at[sl]], r2w.at[sl], sem))
            issue(pltpu.async_copy(p2_in.at[idx2.at[sl]], r2p.at[sl], sem))
        for cp in pending:
            cp.wait()

        def group(g, c2):
            s0, s1, s2 = c2
            outv = zeros
            for i in range(L):
                b = g * L + i
                v0 = r0w[b, :] * r0p[b, :]
                v2 = r2w[pl.ds(b * L, L)] * r2p[pl.ds(b * L, L)]
                t = zeros
                msum = zeros
                for r in range(R):
                    m = r1w[r * C + b, :] * r1p[r * C + b, :]
                    msum = msum + m
                    t = t + v0[r] * m
                sval = jnp.sum(t * v2)
                outv = jnp.where(lanes == i, sval, outv)
                s0 = s0 + v0
                s1 = s1 + msum
                s2 = s2 + v2
            obuf[pl.ds(g * L, L)] = outv
            return (s0, s1, s2)

        s0, s1, s2 = lax.fori_loop(0, C // L, group, (s0, s1, s2))
        pltpu.sync_copy(obuf, out.at[pl.ds(base + o, C)])
        return (s0, s1, s2)

    s0, s1, s2 = lax.fori_loop(0, NCH, subchunk, (zeros, zeros, zeros))

    regbuf[0, :] = s0
    regbuf[1, :] = s1
    regbuf[2, :] = s2
    pltpu.sync_copy(regbuf, partials.at[wid])


@jax.jit
def _tt_lookup(indices, W0, W1, W2, P0, P1, P2):
    mesh = plsc.VectorSubcoreMesh(core_axis_name="c", subcore_axis_name="s")
    f = pl.kernel(
        _sc_body,
        out_type=[
            jax.ShapeDtypeStruct((B,), jnp.float32),
            jax.ShapeDtypeStruct((NW, 3, L), jnp.float32),
        ],
        mesh=mesh,
        compiler_params=pltpu.CompilerParams(
            needs_layout_passes=False, use_tc_tiling_on_sc=False),
        scratch_types=[
            pltpu.VMEM((PER_W, 3), jnp.int32),
            pltpu.VMEM((PER_W,), jnp.int32),
            pltpu.VMEM((PER_W,), jnp.int32),
            pltpu.VMEM((PER_W,), jnp.int32),
            pltpu.VMEM((C * R,), jnp.int32),
            pltpu.VMEM((C, R), jnp.float32),
            pltpu.VMEM((C, R), jnp.float32),
            pltpu.VMEM((C * R, R), jnp.float32),
            pltpu.VMEM((C * R, R), jnp.float32),
            pltpu.VMEM((C * R,), jnp.float32),
            pltpu.VMEM((C * R,), jnp.float32),
            pltpu.VMEM((C,), jnp.float32),
            pltpu.VMEM((3, L), jnp.float32),
            pltpu.SemaphoreType.DMA,
        ],
    )
    return f(indices, W0, W1, W2, P0, P1, P2)


def kernel(indices, W0, W1, W2, P0, P1, P2):
    preds, partials = _tt_lookup(indices, W0, W1,
                                 W2.reshape(R * N), P0, P1, P2.reshape(R * N))
    s = jnp.sum(partials, axis=(0, 2))
    reg = REG_PARA * (jnp.abs(s[0]) / (B * R)
                      + jnp.abs(s[1]) / (B * R * R)
                      + jnp.abs(s[2]) / (B * R))
    return preds, reg
